# Initial kernel scaffold; baseline (speedup 1.0000x reference)
import jax
import jax.numpy as jnp
from jax.experimental import pallas as pl


def kernel(feat, edge_index, efeat):
    raise NotImplementedError("write your pallas kernel here")


# speedup vs baseline: 2.9600x; 2.9600x over previous
import jax
import jax.numpy as jnp
from jax import lax
from jax.experimental import pallas as pl
from jax.experimental.pallas import tpu as pltpu, tpu_sc as plsc

N_NODES = 10000
N_EDGES = 160000
D = 256
DH = 128
NS = 16
E_CHUNK = 80
EDGES_PER_TILE = N_EDGES // NS
CHUNKS_PER_TILE = EDGES_PER_TILE // E_CHUNK
ROWS_PER_TILE = 624
ROWS_TAIL = N_NODES - NS * ROWS_PER_TILE


def _body(f0_hbm, f1_hbm, src_hbm, dst_hbm, efeat_hbm, out_hbm,
          acc, src_v, dst_v, fbuf, ebuf, sem):
    c = lax.axis_index("c")
    s = lax.axis_index("s")

    pltpu.sync_copy(src_hbm.at[s], src_v)
    pltpu.sync_copy(dst_hbm.at[s], dst_v)

    def half(feat_hbm, col0):
        r0 = s * ROWS_PER_TILE
        pltpu.sync_copy(feat_hbm.at[pl.ds(r0, ROWS_PER_TILE)],
                        acc.at[pl.ds(r0, ROWS_PER_TILE)])
        @pl.when(s == NS - 1)
        def _():
            t0 = NS * ROWS_PER_TILE
            pltpu.sync_copy(feat_hbm.at[pl.ds(t0, ROWS_TAIL)],
                            acc.at[pl.ds(t0, ROWS_TAIL)])
        plsc.subcore_barrier()

        def chunk(i, carry):
            off = pl.multiple_of(i * E_CHUNK, 8)
            base = s * EDGES_PER_TILE + off
            pltpu.async_copy(feat_hbm.at[src_v.at[pl.ds(off, E_CHUNK)]],
                             fbuf, sem).wait()
            pltpu.sync_copy(
                efeat_hbm.at[pl.ds(base, E_CHUNK), pl.ds(col0, DH)], ebuf)

            def row(r, rc):
                for j in range(DH // 16):
                    sl = pl.ds(j * 16, 16)
                    v = fbuf[r, sl] + ebuf[r, sl]
                    ebuf[r, sl] = jnp.maximum(v, 0.0)
                return rc
            lax.fori_loop(0, E_CHUNK, row, 0)

            pltpu.sync_copy(ebuf, acc.at[dst_v.at[i]], add=True)
            return carry
        lax.fori_loop(0, CHUNKS_PER_TILE, chunk, 0)

        plsc.subcore_barrier()
        pltpu.sync_copy(acc.at[pl.ds(r0, ROWS_PER_TILE)],
                        out_hbm.at[pl.ds(r0, ROWS_PER_TILE), pl.ds(col0, DH)])
        @pl.when(s == NS - 1)
        def _():
            t0 = NS * ROWS_PER_TILE
            pltpu.sync_copy(acc.at[pl.ds(t0, ROWS_TAIL)],
                            out_hbm.at[pl.ds(t0, ROWS_TAIL), pl.ds(col0, DH)])

    pl.when(c == 0)(lambda: half(f0_hbm, 0))
    pl.when(c == 1)(lambda: half(f1_hbm, DH))


def kernel(feat, edge_index, efeat):
    src2 = edge_index[0].astype(jnp.int32).reshape(NS, EDGES_PER_TILE)
    dst2 = edge_index[1].astype(jnp.int32).reshape(
        NS, CHUNKS_PER_TILE, E_CHUNK)
    f0 = feat[:, :DH]
    f1 = feat[:, DH:]

    run = pl.kernel(
        _body,
        out_type=jax.ShapeDtypeStruct((N_NODES, D), jnp.float32),
        mesh=plsc.VectorSubcoreMesh(core_axis_name="c", subcore_axis_name="s"),
        scratch_types=[
            pltpu.VMEM_SHARED((N_NODES, DH), jnp.float32),
            pltpu.VMEM((EDGES_PER_TILE,), jnp.int32),
            pltpu.VMEM((CHUNKS_PER_TILE, E_CHUNK), jnp.int32),
            pltpu.VMEM((E_CHUNK, DH), jnp.float32),
            pltpu.VMEM((E_CHUNK, DH), jnp.float32),
            pltpu.SemaphoreType.DMA,
        ],
    )
    return run(f0, f1, src2, dst2, efeat)


# speedup vs baseline: 5.5982x; 1.8913x over previous
import jax
import jax.numpy as jnp
from jax import lax
from jax.experimental import pallas as pl
from jax.experimental.pallas import tpu as pltpu, tpu_sc as plsc

N_NODES = 10000
N_EDGES = 160000
D = 256
DH = 128
NS = 16
E_CHUNK = 80
CW = 25
W = 5
EDGES_PER_TILE = N_EDGES // NS
ROWS_PER_TILE = 624
ROWS_TAIL = N_NODES - NS * ROWS_PER_TILE


def _body(fcat_hbm, src4_hbm, dst4_hbm, efeat_hbm, out_hbm,
          acc, src_w, dst_w, fbuf, ebuf, gsem, esem, ssem):
    c = lax.axis_index("c")
    s = lax.axis_index("s")
    col0 = pl.multiple_of(c * DH, DH)
    feat_view = fcat_hbm.at[c]

    r0 = s * ROWS_PER_TILE
    pltpu.sync_copy(feat_view.at[pl.ds(r0, ROWS_PER_TILE)],
                    acc.at[pl.ds(r0, ROWS_PER_TILE)])
    @pl.when(s == NS - 1)
    def _():
        t0 = NS * ROWS_PER_TILE
        pltpu.sync_copy(feat_view.at[pl.ds(t0, ROWS_TAIL)],
                        acc.at[pl.ds(t0, ROWS_TAIL)])
    plsc.subcore_barrier()

    def compute(p):
        fb = fbuf.at[p]
        eb = ebuf.at[p]
        def row(r, rc):
            for j in range(DH // 16):
                sl = pl.ds(j * 16, 16)
                eb[r, sl] = jnp.maximum(fb[r, sl] + eb[r, sl], 0.0)
            return rc
        lax.fori_loop(0, E_CHUNK, row, 0)

    def wave(w, carry):
        pltpu.sync_copy(src4_hbm.at[s, w], src_w)
        pltpu.sync_copy(dst4_hbm.at[s, w], dst_w)

        gd = [None] * CW
        ed = [None] * CW
        sd = [None] * CW

        def issue(i):
            p = i & 1
            base = pl.multiple_of(
                s * EDGES_PER_TILE + (w * CW + i) * E_CHUNK, 8)
            gd[i] = pltpu.async_copy(
                feat_view.at[src_w.at[i]], fbuf.at[p], gsem.at[p])
            ed[i] = pltpu.async_copy(
                efeat_hbm.at[pl.ds(base, E_CHUNK), pl.ds(col0, DH)],
                ebuf.at[p], esem.at[p])

        issue(0)
        for i in range(CW):
            p = i & 1
            if i + 1 < CW:
                if i >= 1:
                    sd[i - 1].wait()
                issue(i + 1)
            gd[i].wait()
            ed[i].wait()
            compute(p)
            sd[i] = pltpu.async_copy(
                ebuf.at[p], acc.at[dst_w.at[i]], ssem.at[p], add=True)
        sd[CW - 2].wait()
        sd[CW - 1].wait()
        return carry

    lax.fori_loop(0, W, wave, 0)

    plsc.subcore_barrier()
    pltpu.sync_copy(acc.at[pl.ds(r0, ROWS_PER_TILE)],
                    out_hbm.at[pl.ds(r0, ROWS_PER_TILE), pl.ds(col0, DH)])
    @pl.when(s == NS - 1)
    def _():
        t0 = NS * ROWS_PER_TILE
        pltpu.sync_copy(acc.at[pl.ds(t0, ROWS_TAIL)],
                        out_hbm.at[pl.ds(t0, ROWS_TAIL), pl.ds(col0, DH)])


def kernel(feat, edge_index, efeat):
    src4 = edge_index[0].astype(jnp.int32).reshape(NS, W, CW, E_CHUNK)
    dst4 = edge_index[1].astype(jnp.int32).reshape(NS, W, CW, E_CHUNK)
    fcat = jnp.stack([feat[:, :DH], feat[:, DH:]])

    run = pl.kernel(
        _body,
        out_type=jax.ShapeDtypeStruct((N_NODES, D), jnp.float32),
        mesh=plsc.VectorSubcoreMesh(core_axis_name="c", subcore_axis_name="s"),
        scratch_types=[
            pltpu.VMEM_SHARED((N_NODES, DH), jnp.float32),
            pltpu.VMEM((CW, E_CHUNK), jnp.int32),
            pltpu.VMEM((CW, E_CHUNK), jnp.int32),
            pltpu.VMEM((2, E_CHUNK, DH), jnp.float32),
            pltpu.VMEM((2, E_CHUNK, DH), jnp.float32),
            pltpu.SemaphoreType.DMA((2,)),
            pltpu.SemaphoreType.DMA((2,)),
            pltpu.SemaphoreType.DMA((2,)),
        ],
    )
    return run(fcat, src4, dst4, efeat)


# speedup vs baseline: 5.5998x; 1.0003x over previous
import jax
import jax.numpy as jnp
from jax import lax
from jax.experimental import pallas as pl
from jax.experimental.pallas import tpu as pltpu, tpu_sc as plsc

N_NODES = 10000
N_EDGES = 160000
D = 256
DH = 128
NS = 16
E_CHUNK = 80
CW = 25
W = 5
EDGES_PER_TILE = N_EDGES // NS
ROWS_PER_TILE = 624
ROWS_TAIL = N_NODES - NS * ROWS_PER_TILE


def _body(feat_hbm, src4_hbm, dst4_hbm, efeat_hbm, out_hbm,
          acc, src_w, dst_w, fbuf, ebuf, gsem, esem, ssem):
    c = lax.axis_index("c")
    s = lax.axis_index("s")
    col0 = pl.multiple_of(c * DH, DH)

    r0 = s * ROWS_PER_TILE
    pltpu.sync_copy(feat_hbm.at[pl.ds(r0, ROWS_PER_TILE), pl.ds(col0, DH)],
                    acc.at[pl.ds(r0, ROWS_PER_TILE)])
    @pl.when(s == NS - 1)
    def _():
        t0 = NS * ROWS_PER_TILE
        pltpu.sync_copy(feat_hbm.at[pl.ds(t0, ROWS_TAIL), pl.ds(col0, DH)],
                        acc.at[pl.ds(t0, ROWS_TAIL)])
    plsc.subcore_barrier()

    def compute(p):
        fb = fbuf.at[p]
        eb = ebuf.at[p]
        def row(r, rc):
            for j in range(DH // 16):
                sl = pl.ds(j * 16, 16)
                eb[r, sl] = jnp.maximum(fb[r, sl] + eb[r, sl], 0.0)
            return rc
        lax.fori_loop(0, E_CHUNK, row, 0)

    def wave(w, carry):
        pltpu.sync_copy(src4_hbm.at[s, w], src_w)
        pltpu.sync_copy(dst4_hbm.at[s, w], dst_w)

        gd = [None] * CW
        ed = [None] * CW
        sd = [None] * CW

        def issue(i):
            p = i & 1
            base = pl.multiple_of(
                s * EDGES_PER_TILE + (w * CW + i) * E_CHUNK, 8)
            gd[i] = pltpu.async_copy(
                feat_hbm.at[src_w.at[i], pl.ds(col0, DH)],
                fbuf.at[p], gsem.at[p])
            ed[i] = pltpu.async_copy(
                efeat_hbm.at[pl.ds(base, E_CHUNK), pl.ds(col0, DH)],
                ebuf.at[p], esem.at[p])

        issue(0)
        for i in range(CW):
            p = i & 1
            if i + 1 < CW:
                if i >= 1:
                    sd[i - 1].wait()
                issue(i + 1)
            gd[i].wait()
            ed[i].wait()
            compute(p)
            sd[i] = pltpu.async_copy(
                ebuf.at[p], acc.at[dst_w.at[i]], ssem.at[p], add=True)
        sd[CW - 2].wait()
        sd[CW - 1].wait()
        return carry

    lax.fori_loop(0, W, wave, 0)

    plsc.subcore_barrier()
    pltpu.sync_copy(acc.at[pl.ds(r0, ROWS_PER_TILE)],
                    out_hbm.at[pl.ds(r0, ROWS_PER_TILE), pl.ds(col0, DH)])
    @pl.when(s == NS - 1)
    def _():
        t0 = NS * ROWS_PER_TILE
        pltpu.sync_copy(acc.at[pl.ds(t0, ROWS_TAIL)],
                        out_hbm.at[pl.ds(t0, ROWS_TAIL), pl.ds(col0, DH)])


def kernel(feat, edge_index, efeat):
    src4 = edge_index[0].astype(jnp.int32).reshape(NS, W, CW, E_CHUNK)
    dst4 = edge_index[1].astype(jnp.int32).reshape(NS, W, CW, E_CHUNK)

    run = pl.kernel(
        _body,
        out_type=jax.ShapeDtypeStruct((N_NODES, D), jnp.float32),
        mesh=plsc.VectorSubcoreMesh(core_axis_name="c", subcore_axis_name="s"),
        scratch_types=[
            pltpu.VMEM_SHARED((N_NODES, DH), jnp.float32),
            pltpu.VMEM((CW, E_CHUNK), jnp.int32),
            pltpu.VMEM((CW, E_CHUNK), jnp.int32),
            pltpu.VMEM((2, E_CHUNK, DH), jnp.float32),
            pltpu.VMEM((2, E_CHUNK, DH), jnp.float32),
            pltpu.SemaphoreType.DMA((2,)),
            pltpu.SemaphoreType.DMA((2,)),
            pltpu.SemaphoreType.DMA((2,)),
        ],
    )
    return run(feat, src4, dst4, efeat)
